# baseline (speedup 1.0000x reference)
import jax
import jax.numpy as jnp
from jax import lax
from jax.experimental import pallas as pl
from jax.experimental.pallas import tpu as pltpu
from jax.experimental.pallas import tpu_sc as plsc

VOCAB = 30522
HIDDEN = 768
BATCH = 64
SEQ = 512
EPS = 1e-12

NC = 2
NS = 16
NW = NC * NS
PPW = SEQ // NW
NCH = 2
BC = BATCH // NCH


def _sc_gather_body(ids_w, wemb, g,
                    idx_v, b0, b1, b2, b3,
                    g0, g1, g2, g3, s0, s1, s2, s3):
    c = lax.axis_index("c")
    s = lax.axis_index("s")
    w = s * NC + c

    pltpu.sync_copy(ids_w.at[w], idx_v)

    bufs = [b0, b1, b2, b3]
    gsems = [g0, g1, g2, g3]
    ssems = [s0, s1, s2, s3]

    def gather(b, k):
        pltpu.async_copy(wemb.at[idx_v.at[pl.ds(b * PPW, PPW)]],
                         bufs[k], gsems[k])

    def step(i, b, k):
        kn = (k + 2) % 4
        pltpu.make_async_copy(wemb.at[pl.ds(0, PPW)], bufs[k], gsems[k]).wait()
        pltpu.async_copy(bufs[k], g.at[b, pl.ds(w * PPW, PPW)], ssems[k])

        @pl.when(b >= 2)
        def _():
            pltpu.make_async_copy(
                bufs[kn], g.at[0, pl.ds(w * PPW, PPW)], ssems[kn]).wait()

        @pl.when(b + 2 < BC)
        def _():
            gather(b + 2, kn)

    gather(0, 0)
    gather(1, 1)

    def loop_body(i, carry):
        step(i, 4 * i, 0)
        step(i, 4 * i + 1, 1)
        step(i, 4 * i + 2, 2)
        step(i, 4 * i + 3, 3)
        return carry

    lax.fori_loop(0, BC // 4, loop_body, 0)

    pltpu.make_async_copy(b2, g.at[0, pl.ds(w * PPW, PPW)], s2).wait()
    pltpu.make_async_copy(b3, g.at[0, pl.ds(w * PPW, PPW)], s3).wait()


def _sc_gather(ids_wc, word_emb):
    kern = pl.kernel(
        _sc_gather_body,
        out_type=jax.ShapeDtypeStruct((BC, SEQ, HIDDEN), jnp.float32),
        mesh=plsc.VectorSubcoreMesh(core_axis_name="c", subcore_axis_name="s"),
        scratch_types=[
            pltpu.VMEM((BC * PPW,), jnp.int32),
            pltpu.VMEM((PPW, HIDDEN), jnp.float32),
            pltpu.VMEM((PPW, HIDDEN), jnp.float32),
            pltpu.VMEM((PPW, HIDDEN), jnp.float32),
            pltpu.VMEM((PPW, HIDDEN), jnp.float32),
            pltpu.SemaphoreType.DMA,
            pltpu.SemaphoreType.DMA,
            pltpu.SemaphoreType.DMA,
            pltpu.SemaphoreType.DMA,
            pltpu.SemaphoreType.DMA,
            pltpu.SemaphoreType.DMA,
            pltpu.SemaphoreType.DMA,
            pltpu.SemaphoreType.DMA,
        ],
    )
    return kern(ids_wc, word_emb)


def _ln_block(g_ref, pos_ref, gam_ref, bet_ref, prev_ref, out_ref):
    x = g_ref[0] + pos_ref[...]
    mean = jnp.mean(x, axis=-1, keepdims=True)
    cx = x - mean
    var = jnp.mean(cx * cx, axis=-1, keepdims=True)
    y = cx * lax.rsqrt(var + EPS)
    out_ref[0] = y * gam_ref[...] + bet_ref[...]


def _tc_ln(g, pos, gam2, bet2, prev, chunk_off):
    return pl.pallas_call(
        _ln_block,
        grid=(BC,),
        in_specs=[
            pl.BlockSpec((1, SEQ, HIDDEN), lambda b: (b, 0, 0)),
            pl.BlockSpec((SEQ, HIDDEN), lambda b: (0, 0)),
            pl.BlockSpec((1, HIDDEN), lambda b: (0, 0)),
            pl.BlockSpec((1, HIDDEN), lambda b: (0, 0)),
            pl.BlockSpec(memory_space=pltpu.MemorySpace.HBM),
        ],
        out_specs=pl.BlockSpec(
            (1, SEQ, HIDDEN), lambda b, _o=chunk_off: (b + _o, 0, 0)),
        out_shape=jax.ShapeDtypeStruct((BATCH, SEQ, HIDDEN), jnp.float32),
        input_output_aliases={4: 0},
    )(g, pos, gam2, bet2, prev)


@jax.jit
def _run(ids_w, word_emb, pos_emb, gam2, bet2):
    out = jnp.zeros((BATCH, SEQ, HIDDEN), jnp.float32)
    for ch in range(NCH):
        g = _sc_gather(ids_w[ch], word_emb)
        out = _tc_ln(g, pos_emb, gam2, bet2, out, ch * BC)
    return out


def kernel(input_ids, word_emb, pos_emb, ln_gamma, ln_beta):
    ids_w = (
        input_ids.reshape(NCH, BC, NW, PPW)
        .transpose(0, 2, 1, 3)
        .reshape(NCH, NW, BC * PPW)
    )
    return _run(ids_w, word_emb, pos_emb,
                ln_gamma.reshape(1, HIDDEN), ln_beta.reshape(1, HIDDEN))


# speedup vs baseline: 1.3642x; 1.3642x over previous
import jax
import jax.numpy as jnp
from jax import lax
from jax.experimental import pallas as pl
from jax.experimental.pallas import tpu as pltpu
from jax.experimental.pallas import tpu_sc as plsc

VOCAB = 30522
HIDDEN = 768
BATCH = 64
SEQ = 512
EPS = 1e-12

NC = 2
NS = 16
NW = NC * NS
PPW = SEQ // NW
NCH = 2
BC = BATCH // NCH


def _sc_gather_body(ids_w, wemb, g,
                    idx_v, b0, b1, b2, b3,
                    g0, g1, g2, g3, s0, s1, s2, s3):
    c = lax.axis_index("c")
    s = lax.axis_index("s")
    w = s * NC + c

    pltpu.sync_copy(ids_w.at[w], idx_v)

    bufs = [b0, b1, b2, b3]
    gsems = [g0, g1, g2, g3]
    ssems = [s0, s1, s2, s3]

    def gather(b, k):
        pltpu.async_copy(wemb.at[idx_v.at[pl.ds(b * PPW, PPW)]],
                         bufs[k], gsems[k])

    def step(i, b, k):
        kn = (k + 2) % 4
        pltpu.make_async_copy(wemb.at[pl.ds(0, PPW)], bufs[k], gsems[k]).wait()
        pltpu.async_copy(bufs[k], g.at[b, pl.ds(w * PPW, PPW)], ssems[k])

        @pl.when(b >= 2)
        def _():
            pltpu.make_async_copy(
                bufs[kn], g.at[0, pl.ds(w * PPW, PPW)], ssems[kn]).wait()

        @pl.when(b + 2 < BC)
        def _():
            gather(b + 2, kn)

    gather(0, 0)
    gather(1, 1)

    def loop_body(i, carry):
        step(i, 4 * i, 0)
        step(i, 4 * i + 1, 1)
        step(i, 4 * i + 2, 2)
        step(i, 4 * i + 3, 3)
        return carry

    lax.fori_loop(0, BC // 4, loop_body, 0)

    pltpu.make_async_copy(b2, g.at[0, pl.ds(w * PPW, PPW)], s2).wait()
    pltpu.make_async_copy(b3, g.at[0, pl.ds(w * PPW, PPW)], s3).wait()


def _sc_gather(ids_wc, word_emb):
    kern = pl.kernel(
        _sc_gather_body,
        out_type=jax.ShapeDtypeStruct((BC, SEQ, HIDDEN), jnp.float32),
        mesh=plsc.VectorSubcoreMesh(core_axis_name="c", subcore_axis_name="s"),
        scratch_types=[
            pltpu.VMEM((BC * PPW,), jnp.int32),
            pltpu.VMEM((PPW, HIDDEN), jnp.float32),
            pltpu.VMEM((PPW, HIDDEN), jnp.float32),
            pltpu.VMEM((PPW, HIDDEN), jnp.float32),
            pltpu.VMEM((PPW, HIDDEN), jnp.float32),
            pltpu.SemaphoreType.DMA,
            pltpu.SemaphoreType.DMA,
            pltpu.SemaphoreType.DMA,
            pltpu.SemaphoreType.DMA,
            pltpu.SemaphoreType.DMA,
            pltpu.SemaphoreType.DMA,
            pltpu.SemaphoreType.DMA,
            pltpu.SemaphoreType.DMA,
        ],
    )
    return kern(ids_wc, word_emb)


def _ln_block(g_ref, pos_ref, gam_ref, bet_ref, *rest):
    out_ref = rest[-1]
    x = g_ref[...] + pos_ref[...][None]
    mean = jnp.mean(x, axis=-1, keepdims=True)
    cx = x - mean
    var = jnp.mean(cx * cx, axis=-1, keepdims=True)
    y = cx * lax.rsqrt(var + EPS)
    out_ref[...] = y * gam_ref[...][None] + bet_ref[...][None]


TB = 4


def _tc_ln(g, pos, gam2, bet2, prev, chunk_off):
    ins = [g, pos, gam2, bet2]
    in_specs = [
        pl.BlockSpec((TB, SEQ, HIDDEN), lambda b: (b, 0, 0)),
        pl.BlockSpec((SEQ, HIDDEN), lambda b: (0, 0)),
        pl.BlockSpec((1, HIDDEN), lambda b: (0, 0)),
        pl.BlockSpec((1, HIDDEN), lambda b: (0, 0)),
    ]
    aliases = {}
    if prev is not None:
        ins.append(prev)
        in_specs.append(pl.BlockSpec(memory_space=pltpu.MemorySpace.HBM))
        aliases = {4: 0}
    off = chunk_off // TB
    return pl.pallas_call(
        _ln_block,
        grid=(BC // TB,),
        in_specs=in_specs,
        out_specs=pl.BlockSpec(
            (TB, SEQ, HIDDEN), lambda b, _o=off: (b + _o, 0, 0)),
        out_shape=jax.ShapeDtypeStruct((BATCH, SEQ, HIDDEN), jnp.float32),
        input_output_aliases=aliases,
    )(*ins)


@jax.jit
def _run(ids_w, word_emb, pos_emb, gam2, bet2):
    out = None
    for ch in range(NCH):
        g = _sc_gather(ids_w[ch], word_emb)
        out = _tc_ln(g, pos_emb, gam2, bet2, out, ch * BC)
    return out


def kernel(input_ids, word_emb, pos_emb, ln_gamma, ln_beta):
    ids_w = (
        input_ids.reshape(NCH, BC, NW, PPW)
        .transpose(0, 2, 1, 3)
        .reshape(NCH, NW, BC * PPW)
    )
    return _run(ids_w, word_emb, pos_emb,
                ln_gamma.reshape(1, HIDDEN), ln_beta.reshape(1, HIDDEN))
